# Initial kernel scaffold; baseline (speedup 1.0000x reference)
import jax
import jax.numpy as jnp
from jax.experimental import pallas as pl


def kernel(seq_items, times, pos, target_items, item_table, time_table, pos_table):
    raise NotImplementedError("write your pallas kernel here")


# speedup vs baseline: 4.0224x; 4.0224x over previous
import functools

import jax
import jax.numpy as jnp
from jax import lax
from jax.experimental import pallas as pl
from jax.experimental.pallas import tpu as pltpu
from jax.experimental.pallas import tpu_sc as plsc

B = 4096
L = 50
D = 64
NSEQ = B * L
NTGT = B

NC = 2
NS = 16
NW = NC * NS
SEQ_PER_W = NSEQ // NW
TGT_PER_W = NTGT // NW
CHUNK = 640
NCHUNKS = SEQ_PER_W // CHUNK


def _emb_body(item_hbm, time_hbm, pos_hbm,
              seq_idx_hbm, time_idx_hbm, pos_idx_hbm, tgt_idx_hbm,
              seq_out, time_out, pos_out, tgt_out,
              seq_idx_v, time_idx_v, pos_idx_v, tgt_idx_v,
              rows0, rows1,
              sem_i, sem_g, sem_s0, sem_s1):
    wid = lax.axis_index("s") * NC + lax.axis_index("c")
    base = wid * SEQ_PER_W
    tbase = wid * TGT_PER_W

    idx_cps = [
        pltpu.async_copy(seq_idx_hbm.at[pl.ds(base, SEQ_PER_W)], seq_idx_v, sem_i),
        pltpu.async_copy(time_idx_hbm.at[pl.ds(base, SEQ_PER_W)], time_idx_v, sem_i),
        pltpu.async_copy(pos_idx_hbm.at[pl.ds(base, SEQ_PER_W)], pos_idx_v, sem_i),
        pltpu.async_copy(tgt_idx_hbm.at[pl.ds(tbase, TGT_PER_W)], tgt_idx_v, sem_i),
    ]
    for cp in idx_cps:
        cp.wait()

    jobs = []
    for tbl, idx_v, out_hbm in ((item_hbm, seq_idx_v, seq_out),
                                (time_hbm, time_idx_v, time_out),
                                (pos_hbm, pos_idx_v, pos_out)):
        for c in range(NCHUNKS):
            jobs.append((tbl,
                         idx_v.at[pl.ds(c * CHUNK, CHUNK)],
                         out_hbm.at[pl.ds(base + c * CHUNK, CHUNK)],
                         CHUNK))
    jobs.append((item_hbm, tgt_idx_v, tgt_out.at[pl.ds(tbase, TGT_PER_W)],
                 TGT_PER_W))

    bufs = (rows0, rows1)
    sems = (sem_s0, sem_s1)
    pending = [None, None]
    for i, (tbl, idxs, outs, cnt) in enumerate(jobs):
        bsel = i % 2
        buf = bufs[bsel] if cnt == CHUNK else bufs[bsel].at[pl.ds(0, cnt)]
        if pending[bsel] is not None:
            pending[bsel].wait()
        pltpu.async_copy(tbl.at[idxs], buf, sem_g).wait()
        pending[bsel] = pltpu.async_copy(buf, outs, sems[bsel])
    for p in pending:
        if p is not None:
            p.wait()


@jax.jit
def _run(seq_idx, time_idx, pos_idx, tgt_idx, item_table, time_table, pos_table):
    mesh = plsc.VectorSubcoreMesh(core_axis_name="c", subcore_axis_name="s")
    f = functools.partial(
        pl.kernel,
        mesh=mesh,
        compiler_params=pltpu.CompilerParams(use_tc_tiling_on_sc=False),
        out_type=[
            jax.ShapeDtypeStruct((NSEQ, D), jnp.float32),
            jax.ShapeDtypeStruct((NSEQ, D), jnp.float32),
            jax.ShapeDtypeStruct((NSEQ, D), jnp.float32),
            jax.ShapeDtypeStruct((NTGT, D), jnp.float32),
        ],
        scratch_types=[
            pltpu.VMEM((SEQ_PER_W,), jnp.int32),
            pltpu.VMEM((SEQ_PER_W,), jnp.int32),
            pltpu.VMEM((SEQ_PER_W,), jnp.int32),
            pltpu.VMEM((TGT_PER_W,), jnp.int32),
            pltpu.VMEM((CHUNK, D), jnp.float32),
            pltpu.VMEM((CHUNK, D), jnp.float32),
            pltpu.SemaphoreType.DMA,
            pltpu.SemaphoreType.DMA,
            pltpu.SemaphoreType.DMA,
            pltpu.SemaphoreType.DMA,
        ],
    )(_emb_body)
    return f(item_table, time_table, pos_table,
             seq_idx, time_idx, pos_idx, tgt_idx)


def kernel(seq_items, times, pos, target_items, item_table, time_table, pos_table):
    seq_idx = seq_items.reshape(-1).astype(jnp.int32)
    time_idx = times.reshape(-1).astype(jnp.int32)
    pos_idx = pos.reshape(-1).astype(jnp.int32)
    tgt_idx = target_items.astype(jnp.int32)
    seq_e, time_e, pos_e, tgt_e = _run(seq_idx, time_idx, pos_idx, tgt_idx,
                                       item_table, time_table, pos_table)
    return (seq_e.reshape(B, L, D), time_e.reshape(B, L, D),
            pos_e.reshape(B, L, D), tgt_e)


# speedup vs baseline: 4.0434x; 1.0052x over previous
import functools

import jax
import jax.numpy as jnp
from jax import lax
from jax.experimental import pallas as pl
from jax.experimental.pallas import tpu as pltpu
from jax.experimental.pallas import tpu_sc as plsc

B = 4096
L = 50
D = 64
NSEQ = B * L
NTGT = B

NC = 2
NS = 16
NW = NC * NS
SEQ_PER_W = NSEQ // NW
TGT_PER_W = NTGT // NW
CHUNK = 400
NCHUNKS = SEQ_PER_W // CHUNK
NB = 3


def _emb_body(item_hbm, time_hbm, pos_hbm,
              seq_idx_hbm, time_idx_hbm, pos_idx_hbm, tgt_idx_hbm,
              seq_out, time_out, pos_out, tgt_out,
              seq_idx_v, time_idx_v, pos_idx_v, tgt_idx_v,
              rows0, rows1, rows2,
              sem_i, gsem0, gsem1, gsem2, ssem0, ssem1, ssem2):
    wid = lax.axis_index("s") * NC + lax.axis_index("c")
    base = wid * SEQ_PER_W
    tbase = wid * TGT_PER_W

    idx_cps = [
        pltpu.async_copy(seq_idx_hbm.at[pl.ds(base, SEQ_PER_W)], seq_idx_v, sem_i),
        pltpu.async_copy(time_idx_hbm.at[pl.ds(base, SEQ_PER_W)], time_idx_v, sem_i),
        pltpu.async_copy(pos_idx_hbm.at[pl.ds(base, SEQ_PER_W)], pos_idx_v, sem_i),
        pltpu.async_copy(tgt_idx_hbm.at[pl.ds(tbase, TGT_PER_W)], tgt_idx_v, sem_i),
    ]
    for cp in idx_cps:
        cp.wait()

    jobs = []
    for tbl, idx_v, out_hbm in ((item_hbm, seq_idx_v, seq_out),
                                (time_hbm, time_idx_v, time_out),
                                (pos_hbm, pos_idx_v, pos_out)):
        for c in range(NCHUNKS):
            jobs.append((tbl,
                         idx_v.at[pl.ds(c * CHUNK, CHUNK)],
                         out_hbm.at[pl.ds(base + c * CHUNK, CHUNK)],
                         CHUNK))
    jobs.append((item_hbm, tgt_idx_v, tgt_out.at[pl.ds(tbase, TGT_PER_W)],
                 TGT_PER_W))

    bufs = (rows0, rows1, rows2)
    gsems = (gsem0, gsem1, gsem2)
    ssems = (ssem0, ssem1, ssem2)
    pend_g = [None] * NB
    pend_s = [None] * NB
    n = len(jobs)
    for t in range(n + NB - 1):
        if t < n:
            b = t % NB
            if pend_s[b] is not None:
                pend_s[b].wait()
            tbl, idxs, outs, cnt = jobs[t]
            buf = bufs[b] if cnt == CHUNK else bufs[b].at[pl.ds(0, cnt)]
            pend_g[b] = (pltpu.async_copy(tbl.at[idxs], buf, gsems[b]),
                         buf, outs)
        j = t - (NB - 1)
        if 0 <= j < n:
            b2 = j % NB
            cp, buf, outs = pend_g[b2]
            cp.wait()
            pend_s[b2] = pltpu.async_copy(buf, outs, ssems[b2])
    for p in pend_s:
        if p is not None:
            p.wait()


@jax.jit
def _run(seq_idx, time_idx, pos_idx, tgt_idx, item_table, time_table, pos_table):
    mesh = plsc.VectorSubcoreMesh(core_axis_name="c", subcore_axis_name="s")
    f = functools.partial(
        pl.kernel,
        mesh=mesh,
        compiler_params=pltpu.CompilerParams(use_tc_tiling_on_sc=False),
        out_type=[
            jax.ShapeDtypeStruct((NSEQ, D), jnp.float32),
            jax.ShapeDtypeStruct((NSEQ, D), jnp.float32),
            jax.ShapeDtypeStruct((NSEQ, D), jnp.float32),
            jax.ShapeDtypeStruct((NTGT, D), jnp.float32),
        ],
        scratch_types=[
            pltpu.VMEM((SEQ_PER_W,), jnp.int32),
            pltpu.VMEM((SEQ_PER_W,), jnp.int32),
            pltpu.VMEM((SEQ_PER_W,), jnp.int32),
            pltpu.VMEM((TGT_PER_W,), jnp.int32),
            pltpu.VMEM((CHUNK, D), jnp.float32),
            pltpu.VMEM((CHUNK, D), jnp.float32),
            pltpu.VMEM((CHUNK, D), jnp.float32),
            pltpu.SemaphoreType.DMA,
            pltpu.SemaphoreType.DMA,
            pltpu.SemaphoreType.DMA,
            pltpu.SemaphoreType.DMA,
            pltpu.SemaphoreType.DMA,
            pltpu.SemaphoreType.DMA,
            pltpu.SemaphoreType.DMA,
        ],
    )(_emb_body)
    return f(item_table, time_table, pos_table,
             seq_idx, time_idx, pos_idx, tgt_idx)


def kernel(seq_items, times, pos, target_items, item_table, time_table, pos_table):
    seq_idx = seq_items.reshape(-1).astype(jnp.int32)
    time_idx = times.reshape(-1).astype(jnp.int32)
    pos_idx = pos.reshape(-1).astype(jnp.int32)
    tgt_idx = target_items.astype(jnp.int32)
    seq_e, time_e, pos_e, tgt_e = _run(seq_idx, time_idx, pos_idx, tgt_idx,
                                       item_table, time_table, pos_table)
    return (seq_e.reshape(B, L, D), time_e.reshape(B, L, D),
            pos_e.reshape(B, L, D), tgt_e)
